# baseline (speedup 1.0000x reference)
import functools

import jax
import jax.numpy as jnp
from jax import lax
from jax.experimental import pallas as pl
from jax.experimental.pallas import tpu as pltpu
from jax.experimental.pallas import tpu_sc as plsc


_N = 256
_NW = 32
_K = 128
_GRP = 32


def _bf_bits(x):
    u = lax.bitcast_convert_type(x, jnp.uint32)
    u = u + jnp.uint32(0x7FFF) + ((u >> 16) & jnp.uint32(1))
    return lax.bitcast_convert_type(u & jnp.uint32(0xFFFF0000), jnp.float32)


def _sc_body(outs_hbm, tgts_hbm, out1, out2,
             orow, trow, nox, noy, txb, tyb, t2a, o2a, m_ot, m_tov):
    wid = lax.axis_index("s") * 2 + lax.axis_index("c")
    bpw = _K // _NW

    def batch_body(b, carry):
        r = wid * bpw + b
        pltpu.sync_copy(outs_hbm.at[r], orow)
        pltpu.sync_copy(tgts_hbm.at[r], trow)

        for c in range(16):
            sl = pl.ds(c * 16, 16)
            ox = orow[pl.ds(c * 16, 16)]
            oy = orow[pl.ds(_N + c * 16, 16)]
            tx = trow[pl.ds(c * 16, 16)]
            ty = trow[pl.ds(_N + c * 16, 16)]
            nox[sl] = -2.0 * _bf_bits(ox)
            noy[sl] = -2.0 * _bf_bits(oy)
            txb[sl] = _bf_bits(tx)
            tyb[sl] = _bf_bits(ty)
            t2a[sl] = tx * tx + ty * ty
            o2a[sl] = ox * ox + oy * oy
            m_ot[sl] = jnp.full((16,), 1e30, jnp.float32)

        def jg_body(g, carry2):
            gsl = pl.ds(g * 16, 16)
            txc = txb[gsl]
            tyc = tyb[gsl]
            t2c = t2a[gsl]
            txk = [jnp.broadcast_to(txc[k:k + 1], (16,)) for k in range(16)]
            tyk = [jnp.broadcast_to(tyc[k:k + 1], (16,)) for k in range(16)]
            t2k = [jnp.broadcast_to(t2c[k:k + 1], (16,)) for k in range(16)]
            macc = [jnp.full((16,), 1e30, jnp.float32) for _ in range(16)]
            for i in range(16):
                isl = pl.ds(i * 16, 16)
                nxi = nox[isl]
                nyi = noy[isl]
                o2i = o2a[isl]
                mi = m_ot[isl]
                for k in range(16):
                    cs = nxi * txk[k] + nyi * tyk[k]
                    mi = jnp.minimum(mi, cs + t2k[k])
                    macc[k] = jnp.minimum(macc[k], cs + o2i)
                m_ot[isl] = mi
            for k in range(16):
                m_tov[g * 16 + k] = macc[k] + t2k[k]
            return carry2

        lax.fori_loop(0, 16, jg_body, 0)
        for c in range(16):
            sl = pl.ds(c * 16, 16)
            m_ot[sl] = m_ot[sl] + o2a[sl]
        pltpu.sync_copy(m_ot, out1.at[r])
        pltpu.sync_copy(m_tov, out2.at[r])
        return carry

    lax.fori_loop(0, _K // _NW, batch_body, 0)


def _sc_epilogue(m1, m2, out_ref):
    d2a = jnp.maximum(m1[...], 0.0)
    s1 = jnp.sum(jnp.sqrt(d2a + 1e-12))
    v = m2[...].reshape(m2.shape[0], _N, 16)
    m_to = jnp.min(v, axis=2)
    d2b = jnp.maximum(m_to, 0.0)
    s2 = jnp.sum(jnp.sqrt(d2b + 1e-12))
    out_ref[0, 0] = s1 + s2


def _tc_body(outs, tgts, out_ref, oxt, oyt, txt, tyt, acc_ref):
    n = outs.shape[1] // 2

    oxt[...] = outs[:, :n].T
    oyt[...] = outs[:, n:].T
    txt[...] = tgts[:, :n].T
    tyt[...] = tgts[:, n:].T

    def bf(x):
        return x.astype(jnp.bfloat16).astype(jnp.float32)

    def pass_sum(ax_ref, ay_ref, bx_ref, by_ref):
        ax = ax_ref[...]
        ay = ay_ref[...]
        a2 = ax * ax + ay * ay
        nax = -2.0 * bf(ax)
        nay = -2.0 * bf(ay)
        acc_ref[...] = jnp.full(acc_ref.shape, 1e30, jnp.float32)

        def grp(g, _):
            base = g * _GRP
            bxg = bx_ref[pl.ds(base, _GRP), :]
            byg = by_ref[pl.ds(base, _GRP), :]
            b2g = bxg * bxg + byg * byg
            bxgb = bf(bxg)
            bygb = bf(byg)
            m0 = acc_ref[...]
            m1 = None
            for k in range(_GRP):
                t1 = nax * bxgb[k:k + 1, :] + b2g[k:k + 1, :]
                t2 = nay * bygb[k:k + 1, :] + t1
                if k % 2 == 0:
                    m0 = jnp.minimum(m0, t2)
                else:
                    m1 = t2 if m1 is None else jnp.minimum(m1, t2)
            acc_ref[...] = jnp.minimum(m0, m1)
            return 0

        jax.lax.fori_loop(0, n // _GRP, grp, 0)
        d2 = jnp.maximum(acc_ref[...] + a2, 0.0)
        return jnp.sum(jnp.sqrt(d2 + 1e-12))

    s_ot = pass_sum(oxt, oyt, txt, tyt)
    s_to = pass_sum(txt, tyt, oxt, oyt)
    out_ref[0, 0] = s_ot + s_to


@functools.partial(jax.jit, static_argnames=())
def kernel(outputs, targets):
    bs, f = outputs.shape
    n = f // 2
    btc = bs - _K

    mesh = plsc.VectorSubcoreMesh(core_axis_name="c", subcore_axis_name="s",
                                  num_cores=2, num_subcores=16)
    sc_min = pl.kernel(
        _sc_body,
        out_type=[
            jax.ShapeDtypeStruct((_K, n), jnp.float32),
            jax.ShapeDtypeStruct((_K, n, 16), jnp.float32),
        ],
        mesh=mesh,
        scratch_types=[
            pltpu.VMEM((2 * n,), jnp.float32),
            pltpu.VMEM((2 * n,), jnp.float32),
            pltpu.VMEM((n,), jnp.float32),
            pltpu.VMEM((n,), jnp.float32),
            pltpu.VMEM((n,), jnp.float32),
            pltpu.VMEM((n,), jnp.float32),
            pltpu.VMEM((n,), jnp.float32),
            pltpu.VMEM((n,), jnp.float32),
            pltpu.VMEM((n,), jnp.float32),
            pltpu.VMEM((n, 16), jnp.float32),
        ],
    )
    m1, m2 = sc_min(outputs[:_K], targets[:_K])

    tc_total = pl.pallas_call(
        _tc_body,
        out_shape=jax.ShapeDtypeStruct((1, 1), jnp.float32),
        in_specs=[pl.BlockSpec((btc, f), lambda: (0, 0))] * 2,
        out_specs=pl.BlockSpec(memory_space=pltpu.SMEM),
        scratch_shapes=[pltpu.VMEM((n, btc), jnp.float32)] * 5,
    )(outputs[_K:], targets[_K:])

    sc_total = pl.pallas_call(
        _sc_epilogue,
        out_shape=jax.ShapeDtypeStruct((1, 1), jnp.float32),
        in_specs=[
            pl.BlockSpec((_K, n), lambda: (0, 0)),
            pl.BlockSpec((_K, n * 16), lambda: (0, 0)),
        ],
        out_specs=pl.BlockSpec(memory_space=pltpu.SMEM),
    )(m1, m2.reshape(_K, n * 16))

    return (tc_total[0, 0] + sc_total[0, 0]) * (0.5 / (bs * n))


# speedup vs baseline: 1.3964x; 1.3964x over previous
import functools

import jax
import jax.numpy as jnp
from jax.experimental import pallas as pl
from jax.experimental.pallas import tpu as pltpu


_GRP = 64


def _body(outs, tgts, out_ref, oxt, oyt, txt, tyt, acc_ref):
    bs = outs.shape[0]
    n = outs.shape[1] // 2

    oxt[...] = outs[:, :n].T
    oyt[...] = outs[:, n:].T
    txt[...] = tgts[:, :n].T
    tyt[...] = tgts[:, n:].T

    def bf(x):
        return x.astype(jnp.bfloat16).astype(jnp.float32)

    def pass_sum(ax_ref, ay_ref, bx_ref, by_ref):
        ax = ax_ref[...]
        ay = ay_ref[...]
        a2 = ax * ax + ay * ay
        nax = -2.0 * bf(ax)
        nay = -2.0 * bf(ay)
        acc_ref[...] = jnp.full(acc_ref.shape, 1e30, jnp.float32)

        def grp(g, _):
            base = g * _GRP
            bxg = bx_ref[pl.ds(base, _GRP), :]
            byg = by_ref[pl.ds(base, _GRP), :]
            b2g = bxg * bxg + byg * byg
            bxgb = bf(bxg)
            bygb = bf(byg)
            m0 = acc_ref[...]
            m1 = None
            for k in range(_GRP):
                t1 = nax * bxgb[k:k + 1, :] + b2g[k:k + 1, :]
                t2 = nay * bygb[k:k + 1, :] + t1
                if k % 2 == 0:
                    m0 = jnp.minimum(m0, t2)
                else:
                    m1 = t2 if m1 is None else jnp.minimum(m1, t2)
            acc_ref[...] = jnp.minimum(m0, m1)
            return 0

        jax.lax.fori_loop(0, n // _GRP, grp, 0)
        d2 = jnp.maximum(acc_ref[...] + a2, 0.0)
        return jnp.sum(jnp.sqrt(d2 + 1e-12))

    s_ot = pass_sum(oxt, oyt, txt, tyt)
    s_to = pass_sum(txt, tyt, oxt, oyt)
    out_ref[0, 0] = s_ot + s_to


@functools.partial(jax.jit, static_argnames=())
def kernel(outputs, targets):
    bs, f = outputs.shape
    n = f // 2

    total = pl.pallas_call(
        _body,
        out_shape=jax.ShapeDtypeStruct((1, 1), jnp.float32),
        in_specs=[pl.BlockSpec((bs, f), lambda: (0, 0))] * 2,
        out_specs=pl.BlockSpec(memory_space=pltpu.SMEM),
        scratch_shapes=[pltpu.VMEM((n, bs), jnp.float32)] * 5,
    )(outputs, targets)

    return total[0, 0] * (0.5 / (bs * n))


# speedup vs baseline: 1.4274x; 1.0222x over previous
import functools

import jax
import jax.numpy as jnp
from jax.experimental import pallas as pl
from jax.experimental.pallas import tpu as pltpu


_GRP = 32


def _body(outs, tgts, out_ref, soxr, soyr, btxr, btyr, o2r, t2r,
          acc1_ref, acc2_ref):
    bs = outs.shape[0]
    n = outs.shape[1] // 2

    def bf(x):
        return x.astype(jnp.bfloat16).astype(jnp.float32)

    ox = outs[:, :n].T
    oy = outs[:, n:].T
    tx = tgts[:, :n].T
    ty = tgts[:, n:].T
    o2r[...] = ox * ox + oy * oy
    t2r[...] = tx * tx + ty * ty
    soxr[...] = -2.0 * bf(ox)
    soyr[...] = -2.0 * bf(oy)
    btxr[...] = bf(tx)
    btyr[...] = bf(ty)
    sox = soxr[...]
    soy = soyr[...]
    btx = btxr[...]
    bty = btyr[...]
    o2 = o2r[...]
    t2 = t2r[...]
    acc1_ref[...] = jnp.full(acc1_ref.shape, 1e30, jnp.float32)
    acc2_ref[...] = jnp.full(acc2_ref.shape, 1e30, jnp.float32)

    def grp(g, _):
        sl = pl.ds(g * _GRP, _GRP)
        btxg = btxr[sl, :]
        btyg = btyr[sl, :]
        t2g = t2r[sl, :]
        soxg = soxr[sl, :]
        soyg = soyr[sl, :]
        o2g = o2r[sl, :]
        m1a = acc1_ref[...]
        m2a = acc2_ref[...]
        m1b = None
        m2b = None
        for k in range(_GRP):
            u1 = sox * btxg[k:k + 1, :] + t2g[k:k + 1, :]
            u1 = soy * btyg[k:k + 1, :] + u1
            u2 = btx * soxg[k:k + 1, :] + o2g[k:k + 1, :]
            u2 = bty * soyg[k:k + 1, :] + u2
            if k % 2 == 0:
                m1a = jnp.minimum(m1a, u1)
                m2a = jnp.minimum(m2a, u2)
            else:
                m1b = u1 if m1b is None else jnp.minimum(m1b, u1)
                m2b = u2 if m2b is None else jnp.minimum(m2b, u2)
        acc1_ref[...] = jnp.minimum(m1a, m1b)
        acc2_ref[...] = jnp.minimum(m2a, m2b)
        return 0

    jax.lax.fori_loop(0, n // _GRP, grp, 0)
    d2_ot = jnp.maximum(acc1_ref[...] + o2, 0.0)
    d2_to = jnp.maximum(acc2_ref[...] + t2, 0.0)
    out_ref[0, 0] = (jnp.sum(jnp.sqrt(d2_ot + 1e-12))
                     + jnp.sum(jnp.sqrt(d2_to + 1e-12)))


@functools.partial(jax.jit, static_argnames=())
def kernel(outputs, targets):
    bs, f = outputs.shape
    n = f // 2

    total = pl.pallas_call(
        _body,
        out_shape=jax.ShapeDtypeStruct((1, 1), jnp.float32),
        in_specs=[pl.BlockSpec((bs, f), lambda: (0, 0))] * 2,
        out_specs=pl.BlockSpec(memory_space=pltpu.SMEM),
        scratch_shapes=[pltpu.VMEM((n, bs), jnp.float32)] * 8,
    )(outputs, targets)

    return total[0, 0] * (0.5 / (bs * n))
